# baseline (device time: 700849 ns/iter reference)
import jax
import jax.numpy as jnp
from jax import lax
from jax.experimental import pallas as pl
from jax.experimental.pallas import tpu as pltpu

N = 16


def kernel(x, Win0, Wout0, Win1, Wout1, Win2, Wout2):
    b, d = x.shape
    h = Win0.shape[1]

    def body(x_ref, win0, wout0, win1, wout1, win2, wout2, out_ref,
             xg, pg, rs_buf, ag_send, ag_recv, rs_send, rs_recv):
        my = lax.axis_index("i")
        left = (my + N - 1) % N
        right = (my + 1) % N

        barrier = pltpu.get_barrier_semaphore()
        for nbr in (left, right):
            pl.semaphore_signal(barrier, inc=1, device_id=(nbr,),
                                device_id_type=pl.DeviceIdType.MESH)
        pl.semaphore_wait(barrier, 2)

        xg[my] = x_ref[...]

        layers = ((win0, wout0), (win1, wout1), (win2, wout2))
        for l, (win, wout) in enumerate(layers):
            for hop in range(N - 1):
                slot = (my - hop) % N
                rdma = pltpu.make_async_remote_copy(
                    src_ref=xg.at[slot],
                    dst_ref=xg.at[slot],
                    send_sem=ag_send.at[hop],
                    recv_sem=ag_recv.at[hop],
                    device_id=(right,),
                    device_id_type=pl.DeviceIdType.MESH,
                )
                rdma.start()
                rdma.wait()

            for blk in range(N):
                hid = jnp.maximum(
                    jnp.dot(xg[blk], win[...],
                            preferred_element_type=jnp.float32),
                    0.0,
                )
                pg[blk] = jnp.dot(hid, wout[...],
                                  preferred_element_type=jnp.float32)

            for s in range(N - 1):
                send_c = (my - s - 1) % N
                recv_c = (my - s - 2) % N
                rdma = pltpu.make_async_remote_copy(
                    src_ref=pg.at[send_c],
                    dst_ref=rs_buf.at[s],
                    send_sem=rs_send.at[s],
                    recv_sem=rs_recv.at[s],
                    device_id=(right,),
                    device_id_type=pl.DeviceIdType.MESH,
                )
                rdma.start()
                rdma.wait()
                pg[recv_c] = pg[recv_c] + rs_buf[s]

            if l < len(layers) - 1:
                xg[my] = pg[my]

        out_ref[...] = pg[my]

    return pl.pallas_call(
        body,
        out_shape=jax.ShapeDtypeStruct((b, d), jnp.float32),
        in_specs=[pl.BlockSpec(memory_space=pltpu.VMEM)] * 7,
        out_specs=pl.BlockSpec(memory_space=pltpu.VMEM),
        scratch_shapes=[
            pltpu.VMEM((N, b, d), jnp.float32),
            pltpu.VMEM((N, b, d), jnp.float32),
            pltpu.VMEM((N - 1, b, d), jnp.float32),
            pltpu.SemaphoreType.DMA((N - 1,)),
            pltpu.SemaphoreType.DMA((N - 1,)),
            pltpu.SemaphoreType.DMA((N - 1,)),
            pltpu.SemaphoreType.DMA((N - 1,)),
        ],
        compiler_params=pltpu.CompilerParams(collective_id=0),
    )(x, Win0, Wout0, Win1, Wout1, Win2, Wout2)


# device time: 284172 ns/iter; 2.4663x vs baseline; 2.4663x over previous
import jax
import jax.numpy as jnp
from jax import lax
from jax.experimental import pallas as pl
from jax.experimental.pallas import tpu as pltpu

N = 16


def kernel(x, Win0, Wout0, Win1, Wout1, Win2, Wout2):
    b, d = x.shape
    h = Win0.shape[1]
    hh = h // 2
    assert hh == d, "slot layout assumes square (d, hh) halves"
    f32 = jnp.float32
    bf16 = jnp.bfloat16

    def body(x_ref, win0, wout0, win1, wout1, win2, wout2, out_ref,
             wr, wl, xcur, xbf, acc,
             send_r, recv_r, send_l, recv_l, credit_r, credit_l):
        my = lax.axis_index("i")
        left = (my + N - 1) % N
        right = (my + 1) % N

        barrier = pltpu.get_barrier_semaphore()
        for nbr in (left, right):
            pl.semaphore_signal(barrier, inc=1, device_id=(nbr,),
                                device_id_type=pl.DeviceIdType.MESH)
        pl.semaphore_wait(barrier, 2)

        xcur[...] = x_ref[...]

        layers = ((win0, wout0), (win1, wout1), (win2, wout2))
        for l, (win, wout) in enumerate(layers):
            wr[0, 0] = win[:, :hh].astype(bf16)
            wr[0, 1] = wout[:hh, :].astype(bf16)
            wl[0, 0] = win[:, hh:].astype(bf16)
            wl[0, 1] = wout[hh:, :].astype(bf16)
            xbf[...] = xcur[...].astype(bf16)

            if l > 0:
                pl.semaphore_wait(credit_r, 1)
                pl.semaphore_wait(credit_l, 1)

            r_desc = []
            l_desc = []
            for s in range(N - 1):
                r_desc.append(pltpu.make_async_remote_copy(
                    src_ref=wr.at[s], dst_ref=wr.at[s + 1],
                    send_sem=send_r.at[s], recv_sem=recv_r.at[s],
                    device_id=(right,), device_id_type=pl.DeviceIdType.MESH))
                l_desc.append(pltpu.make_async_remote_copy(
                    src_ref=wl.at[s], dst_ref=wl.at[s + 1],
                    send_sem=send_l.at[s], recv_sem=recv_l.at[s],
                    device_id=(left,), device_id_type=pl.DeviceIdType.MESH))
            r_desc[0].start()
            l_desc[0].start()

            hid0 = jnp.maximum(
                jnp.dot(xcur[...], win[...], preferred_element_type=f32), 0.0)
            acc[...] = jnp.dot(hid0, wout[...], preferred_element_type=f32)

            def add_contrib(wbuf, s):
                hid = jnp.maximum(
                    jnp.dot(xbf[...], wbuf[s, 0],
                            preferred_element_type=f32), 0.0).astype(bf16)
                acc[...] = acc[...] + jnp.dot(hid, wbuf[s, 1],
                                              preferred_element_type=f32)

            for s in range(1, N):
                r_desc[s - 1].wait()
                l_desc[s - 1].wait()
                if s < N - 1:
                    r_desc[s].start()
                    l_desc[s].start()
                add_contrib(wr, s)
                add_contrib(wl, s)

            if l < len(layers) - 1:
                pl.semaphore_signal(credit_r, inc=1, device_id=(left,),
                                    device_id_type=pl.DeviceIdType.MESH)
                pl.semaphore_signal(credit_l, inc=1, device_id=(right,),
                                    device_id_type=pl.DeviceIdType.MESH)
                xcur[...] = acc[...]

        out_ref[...] = acc[...]

    return pl.pallas_call(
        body,
        out_shape=jax.ShapeDtypeStruct((b, d), jnp.float32),
        in_specs=[pl.BlockSpec(memory_space=pltpu.VMEM)] * 7,
        out_specs=pl.BlockSpec(memory_space=pltpu.VMEM),
        scratch_shapes=[
            pltpu.VMEM((N, 2, d, hh), bf16),
            pltpu.VMEM((N, 2, d, hh), bf16),
            pltpu.VMEM((b, d), f32),
            pltpu.VMEM((b, d), bf16),
            pltpu.VMEM((b, d), f32),
            pltpu.SemaphoreType.DMA((N - 1,)),
            pltpu.SemaphoreType.DMA((N - 1,)),
            pltpu.SemaphoreType.DMA((N - 1,)),
            pltpu.SemaphoreType.DMA((N - 1,)),
            pltpu.SemaphoreType.REGULAR,
            pltpu.SemaphoreType.REGULAR,
        ],
        compiler_params=pltpu.CompilerParams(collective_id=0),
    )(x, Win0, Wout0, Win1, Wout1, Win2, Wout2)
